# baseline (speedup 1.0000x reference)
import functools

import jax
import jax.numpy as jnp
from jax import lax
from jax.experimental import pallas as pl
from jax.experimental.pallas import tpu as pltpu
from jax.experimental.pallas import tpu_sc as plsc

NC = 2
NS = 16
NW = NC * NS
CH = 128


def _sc_scatter_call(d, nseg, n_pad):
    rpz = n_pad // NS
    zfull = rpz // CH
    zrem = rpz % CH

    mesh = plsc.VectorSubcoreMesh(
        core_axis_name="c", subcore_axis_name="s", num_cores=NC,
        num_subcores=NS)

    @functools.partial(
        pl.kernel,
        mesh=mesh,
        out_type=jax.ShapeDtypeStruct((NC, n_pad, d), jnp.float32),
        scratch_types=[
            pltpu.VMEM((nseg, CH), jnp.int32),
            pltpu.VMEM((nseg, CH), jnp.int32),
            pltpu.VMEM((CH, d), jnp.float32),
            pltpu.VMEM_SHARED((n_pad, d), jnp.float32),
            pltpu.SemaphoreType.DMA,
        ],
    )
    def scatter_kernel(rows_hbm, src_hbm, dst_hbm, out_hbm,
                       src_v, dst_v, rows_v, acc_sh, sem):
        c = lax.axis_index("c")
        s = lax.axis_index("s")

        zero16 = jnp.zeros((16,), jnp.float32)

        def zrow(i, carry):
            for j in range(d // 16):
                rows_v[i, pl.ds(j * 16, 16)] = zero16
            return carry

        lax.fori_loop(0, CH, zrow, 0)
        for k in range(zfull):
            pltpu.sync_copy(rows_v, acc_sh.at[pl.ds(s * rpz + k * CH, CH)])
        if zrem:
            pltpu.sync_copy(
                rows_v.at[pl.ds(0, zrem)],
                acc_sh.at[pl.ds(s * rpz + zfull * CH, zrem)])
        plsc.subcore_barrier()

        def run_seg(slot):
            pltpu.sync_copy(src_hbm.at[s, slot], src_v)
            pltpu.sync_copy(dst_hbm.at[s, slot], dst_v)

            def step(j, carry):
                pltpu.async_copy(rows_hbm.at[src_v.at[j]], rows_v,
                                 sem).wait()
                pltpu.sync_copy(rows_v, acc_sh.at[dst_v.at[j]], add=True)
                return carry

            lax.fori_loop(0, nseg, step, 0)

        @pl.when(c == 0)
        def _():
            run_seg(0)
            run_seg(1)

        @pl.when(c == 1)
        def _():
            run_seg(2)

        plsc.subcore_barrier()

        pltpu.sync_copy(acc_sh.at[pl.ds(s * rpz, rpz)],
                        out_hbm.at[c, pl.ds(s * rpz, rpz)])

    return scatter_kernel


def _combine_matmul_relu_call(p, w, rows_blk):
    _, n, d = p.shape

    def body(p_ref, w_ref, o_ref):
        agg = p_ref[0] + p_ref[1]
        o_ref[...] = jnp.maximum(
            jnp.dot(agg, w_ref[...], preferred_element_type=jnp.float32), 0.0)

    return pl.pallas_call(
        body,
        grid=(n // rows_blk,),
        in_specs=[
            pl.BlockSpec((NC, rows_blk, d), lambda i: (0, i, 0)),
            pl.BlockSpec((d, d), lambda i: (0, 0)),
        ],
        out_specs=pl.BlockSpec((rows_blk, d), lambda i: (i, 0)),
        out_shape=jax.ShapeDtypeStruct((n, d), jnp.float32),
    )(p, w)


def kernel(A, X, W1, W2):
    x = X[0]
    n, d = x.shape
    e = A.shape[1]

    nseg = -(-e // (NS * 3 * CH))
    e_pad = NS * 3 * nseg * CH
    n_pad = -(-(n + 1) // 128) * 128

    trash = n + jnp.arange(e_pad - e, dtype=jnp.int32) % (n_pad - n)
    src = jnp.concatenate(
        [A[0], jnp.zeros((e_pad - e,), jnp.int32)]).reshape(NS, 3, nseg, CH)
    dst = jnp.concatenate([A[1], trash]).reshape(NS, 3, nseg, CH)

    scatter = _sc_scatter_call(d, nseg, n_pad)

    blk = n_pad // 8
    p1 = scatter(x, src, dst)
    h1 = _combine_matmul_relu_call(p1, W1, blk)
    p2 = scatter(h1, src, dst)
    out = _combine_matmul_relu_call(p2, W2, blk)
    return out[None, :n, :]
